# baseline (speedup 1.0000x reference)
import jax
import jax.numpy as jnp
from jax import lax
from jax.experimental import pallas as pl
from jax.experimental.pallas import tpu as pltpu
from jax.experimental.pallas import tpu_sc as plsc

B, F, P = 1024, 26, 40
VOCAB, EMB = 100000, 32

N = B * F * P
NC, NS = 2, 16
NW = NC * NS
BW = B // NW
SR = BW * P
NBUF = 2


def _sc_gather(table, idx):
    mesh = plsc.VectorSubcoreMesh(core_axis_name="c", subcore_axis_name="s")

    @pl.kernel(
        out_type=jax.ShapeDtypeStruct((N, EMB), jnp.float32),
        mesh=mesh,
        scratch_types=[
            [pltpu.VMEM((SR,), jnp.int32) for _ in range(NBUF)],
            [pltpu.VMEM((SR, EMB), jnp.float32) for _ in range(NBUF)],
            [pltpu.SemaphoreType.DMA for _ in range(NBUF)],
            [pltpu.SemaphoreType.DMA for _ in range(NBUF)],
            [pltpu.SemaphoreType.DMA for _ in range(NBUF)],
        ],
        compiler_params=pltpu.CompilerParams(use_tc_tiling_on_sc=False,
                                             needs_layout_passes=False),
    )
    def k(table_hbm, idx_hbm, out_hbm, idx_v, rows, isem, gsem, ssem):
        wid = lax.axis_index("s") * NC + lax.axis_index("c")
        b0 = wid * BW

        def stage(f, b):
            return pltpu.make_async_copy(
                idx_hbm.at[pl.ds((f * B + b0) * P, SR)], idx_v[b], isem[b])

        def gather(b):
            return pltpu.make_async_copy(table_hbm.at[idx_v[b]], rows[b],
                                         gsem[b])

        def store(f, b):
            return pltpu.make_async_copy(
                rows[b], out_hbm.at[pl.ds((f * B + b0) * P, SR)], ssem[b])

        for b in range(NBUF):
            stage(b, b).start()

        def body(u, _):
            for b in range(NBUF):
                f = u * NBUF + b
                stage(f, b).wait()

                @pl.when(u > 0)
                def _drain():
                    store(f, b).wait()

                gather(b).start()
            for b in range(NBUF):
                f = u * NBUF + b
                gather(b).wait()

                @pl.when(f + NBUF < F)
                def _prefetch():
                    stage(f + NBUF, b).start()

                store(f, b).start()
            return _

        lax.fori_loop(0, F // NBUF, body, None)
        for b in range(NBUF):
            store(b, b).wait()

    return k(table, idx)


def kernel(feature, table):
    idx = feature.transpose(1, 0, 2).reshape(N)
    out = _sc_gather(table, idx)
    return out.reshape(F, B, P * EMB).transpose(1, 0, 2)


# speedup vs baseline: 1.0051x; 1.0051x over previous
import jax
import jax.numpy as jnp
from jax import lax
from jax.experimental import pallas as pl
from jax.experimental.pallas import tpu as pltpu
from jax.experimental.pallas import tpu_sc as plsc

B, F, P = 1024, 26, 40
VOCAB, EMB = 100000, 32

N = B * F * P
NC, NS = 2, 16
NW = NC * NS
BW = B // NW
SR = BW * P
NBUF = 2


def _sc_gather(table, idx):
    mesh = plsc.VectorSubcoreMesh(core_axis_name="c", subcore_axis_name="s")

    @pl.kernel(
        out_type=jax.ShapeDtypeStruct((N, EMB), jnp.float32),
        mesh=mesh,
        scratch_types=[
            pltpu.VMEM((F, SR), jnp.int32),
            [pltpu.VMEM((SR, EMB), jnp.float32) for _ in range(NBUF)],
            [pltpu.SemaphoreType.DMA for _ in range(NBUF)],
            [pltpu.SemaphoreType.DMA for _ in range(NBUF)],
        ],
        compiler_params=pltpu.CompilerParams(use_tc_tiling_on_sc=False,
                                             needs_layout_passes=False),
    )
    def k(table_hbm, idx_hbm, out_hbm, idx_v, rows, gsem, ssem):
        wid = lax.axis_index("s") * NC + lax.axis_index("c")
        b0 = wid * BW

        pltpu.sync_copy(idx_hbm.at[pl.ds(0, F), pl.ds(b0 * P, SR)], idx_v)

        def gather(f, b):
            return pltpu.make_async_copy(table_hbm.at[idx_v.at[f]], rows[b],
                                         gsem[b])

        def store(f, b):
            return pltpu.make_async_copy(
                rows[b], out_hbm.at[pl.ds((f * B + b0) * P, SR)], ssem[b])

        def body(u, _):
            for b in range(NBUF):
                f = u * NBUF + b

                @pl.when(u > 0)
                def _drain():
                    store(f, b).wait()

                gather(f, b).start()
            for b in range(NBUF):
                f = u * NBUF + b
                gather(f, b).wait()
                store(f, b).start()
            return _

        lax.fori_loop(0, F // NBUF, body, None)
        for b in range(NBUF):
            store(b, b).wait()

    return k(table, idx)


def kernel(feature, table):
    idx = feature.transpose(1, 0, 2).reshape(F, B * P)
    out = _sc_gather(table, idx)
    return out.reshape(F, B, P * EMB).transpose(1, 0, 2)
